# baseline (speedup 1.0000x reference)
import functools

import jax
import jax.numpy as jnp
from jax import lax
from jax.experimental import pallas as pl
from jax.experimental.pallas import tpu as pltpu
from jax.experimental.pallas import tpu_sc as plsc

_B, _H, _W = 4, 256, 256
_NBINS = 64
_DLO = 104
_NTILES = 32
_JPT = _W // 8
_MAGIC = 12582912.0


def _sc_body(up_hbm, left_hbm, right_hbm, out_hbm,
             ublk, lblk, rblk, acc_l, acc_r, outv, sem_u, sem_lr):
  nc = 2
  wid = lax.axis_index("s") * nc + lax.axis_index("c")
  b = wid // 8
  j0 = (wid % 8) * _JPT

  cp_u = pltpu.async_copy(
      up_hbm.at[pl.ds(b * _H, _H), pl.ds(j0, _JPT)], ublk, sem_u)
  cp_l = pltpu.async_copy(
      left_hbm.at[pl.ds(b * _H + j0, _JPT), pl.ds(_DLO, _NBINS)], lblk,
      sem_lr)
  cp_r = pltpu.async_copy(
      right_hbm.at[pl.ds(b * _H + j0, _JPT), pl.ds(_DLO, _NBINS)], rblk,
      sem_lr)

  z = jnp.zeros((16,), jnp.float32)

  def zero_k(k, carry):
    o = k * 16
    acc_l[pl.ds(o, 16)] = z
    acc_r[pl.ds(o, 16)] = z
    return carry

  lax.fori_loop(0, _JPT * _NBINS // 16, zero_k, 0, unroll=8)
  cp_u.wait()

  iota = lax.iota(jnp.int32, 16)
  jbase = (iota * _NBINS, (iota + 16) * _NBINS)
  shift = tuple((_MAGIC + float(_DLO)) - jb.astype(jnp.float32)
                for jb in jbase)
  lo = jbase
  hi = tuple(jb + (_NBINS - 1) for jb in jbase)

  def do_row(i, acc, val):
    for g in range(2):
      u = ublk[i, pl.ds(g * 16, 16)]
      su = u * 50.0 + 110.0
      fidx = ((su + _MAGIC) - shift[g]).astype(jnp.int32)
      fidx = jnp.minimum(jnp.maximum(fidx, lo[g]), hi[g])
      plsc.store_scatter(acc, [fidx], val, mask=(u >= 0.0235))

  def right_row(k, carry):
    kf = jnp.full((16,), k, jnp.int32).astype(jnp.float32)
    do_row(k + 129, acc_r, (kf + 1.0) / 60.0)
    return carry

  def left_row(k, carry):
    kf = jnp.full((16,), k, jnp.int32).astype(jnp.float32)
    do_row(127 - k, acc_l, (kf + 1.0) / 60.0)
    return carry

  lax.fori_loop(0, 127, right_row, 0, unroll=4)
  lax.fori_loop(0, 128, left_row, 0, unroll=4)

  cp_l.wait()
  cp_r.wait()

  def loss_j(j, s):
    o = j * _NBINS
    for v in range(_NBINS // 16):
      sl = pl.ds(o + v * 16, 16)
      a_l = acc_l[sl]
      a_r = acc_r[sl]
      lv = lblk[j, pl.ds(v * 16, 16)]
      rv = rblk[j, pl.ds(v * 16, 16)]
      dl = jnp.abs(a_l - lv)
      s = s + jnp.where((dl < 0.2) & (a_l != 0.0), dl, 0.0)
      dr = jnp.abs(a_r - rv)
      s = s + jnp.where((dr < 0.2) & (a_r != 0.0), dr, 0.0)
    return s

  total = lax.fori_loop(0, _JPT, loss_j, jnp.zeros((16,), jnp.float32),
                        unroll=2)
  outv[pl.ds(0, 16)] = total
  pltpu.sync_copy(outv, out_hbm.at[wid])


@functools.partial(
    pl.kernel,
    out_type=jax.ShapeDtypeStruct((_NTILES, 16), jnp.float32),
    mesh=plsc.VectorSubcoreMesh(core_axis_name="c", subcore_axis_name="s"),
    compiler_params=pltpu.CompilerParams(
        use_tc_tiling_on_sc=False, needs_layout_passes=False),
    scratch_types=[
        pltpu.VMEM((_H, _JPT), jnp.float32),
        pltpu.VMEM((_JPT, _NBINS), jnp.float32),
        pltpu.VMEM((_JPT, _NBINS), jnp.float32),
        pltpu.VMEM((_JPT * _NBINS,), jnp.float32),
        pltpu.VMEM((_JPT * _NBINS,), jnp.float32),
        pltpu.VMEM((16,), jnp.float32),
        pltpu.SemaphoreType.DMA,
        pltpu.SemaphoreType.DMA,
    ],
)
def _sc_loss_partials(up_hbm, left_hbm, right_hbm, out_hbm, *scratch):
  _sc_body(up_hbm, left_hbm, right_hbm, out_hbm, *scratch)


def _tc_reduce_body(p_ref, o_ref):
  o_ref[0, 0] = jnp.sum(p_ref[...]) * (1.0 / float(_B * _H * _W))


@jax.jit
def kernel(up, left, right):
  u2 = up.reshape(_B * _H, _W)
  l2 = left.reshape(_B * _H, _W)
  r2 = right.reshape(_B * _H, _W)
  partials = _sc_loss_partials(u2, l2, r2)
  out = pl.pallas_call(
      _tc_reduce_body,
      out_shape=jax.ShapeDtypeStruct((1, 1), jnp.float32),
      out_specs=pl.BlockSpec(memory_space=pltpu.SMEM),
  )(partials)
  return out[0, 0]


# speedup vs baseline: 1.1172x; 1.1172x over previous
import functools

import jax
import jax.numpy as jnp
from jax import lax
from jax.experimental import pallas as pl
from jax.experimental.pallas import tpu as pltpu
from jax.experimental.pallas import tpu_sc as plsc

_B, _H, _W = 4, 256, 256
_NBINS = 64
_DLO = 104
_NTILES = 32
_JPT = _W // 8
_MAGIC = 12582912.0


def _sc_body(up_hbm, left_hbm, right_hbm, out_hbm,
             ublk, lblk, rblk, acc_l, acc_r, fidx_tbl, outv, sem_u, sem_lr):
  nc = 2
  wid = lax.axis_index("s") * nc + lax.axis_index("c")
  b = wid // 8
  j0 = (wid % 8) * _JPT

  cp_u = pltpu.async_copy(
      up_hbm.at[pl.ds(b * _H, _H), pl.ds(j0, _JPT)], ublk, sem_u)
  cp_l = pltpu.async_copy(
      left_hbm.at[pl.ds(b * _H + j0, _JPT), pl.ds(_DLO, _NBINS)], lblk,
      sem_lr)
  cp_r = pltpu.async_copy(
      right_hbm.at[pl.ds(b * _H + j0, _JPT), pl.ds(_DLO, _NBINS)], rblk,
      sem_lr)

  z = jnp.zeros((16,), jnp.float32)

  def zero_k(k, carry):
    o = k * 16
    acc_l[pl.ds(o, 16)] = z
    acc_r[pl.ds(o, 16)] = z
    return carry

  lax.fori_loop(0, _JPT * _NBINS // 16, zero_k, 0)
  cp_u.wait()

  iota = lax.iota(jnp.int32, 16)
  jbase = (iota * _NBINS, (iota + 16) * _NBINS)
  shift = tuple((_MAGIC + float(_DLO)) - jb.astype(jnp.float32)
                for jb in jbase)
  lo = jbase
  hi = tuple(jb + (_NBINS - 1) for jb in jbase)

  dump = jnp.full((16,), _JPT * _NBINS, jnp.int32) + iota

  @plsc.parallel_loop(0, _H, unroll=4)
  def phase_a(i):
    for g in range(2):
      u = ublk[i, pl.ds(g * 16, 16)]
      su = u * 50.0 + 110.0
      fidx = ((su + _MAGIC) - shift[g]).astype(jnp.int32)
      fidx = jnp.minimum(jnp.maximum(fidx, lo[g]), hi[g])
      fidx_tbl[pl.ds(i * _JPT + g * 16, 16)] = jnp.where(
          u >= 0.0235, fidx, dump)

  def row_pair(k, carry):
    kf = jnp.full((16,), k, jnp.int32).astype(jnp.float32)
    val = (kf + 1.0) / 60.0
    o_r = (k + 129) * _JPT
    o_l = (127 - k) * _JPT
    for g in range(2):
      plsc.store_scatter(acc_r, [fidx_tbl[pl.ds(o_r + g * 16, 16)]], val)
      plsc.store_scatter(acc_l, [fidx_tbl[pl.ds(o_l + g * 16, 16)]], val)
    return carry

  lax.fori_loop(0, 127, row_pair, 0)
  v0 = jnp.full((16,), 128.0, jnp.float32) / 60.0
  for g in range(2):
    plsc.store_scatter(acc_l, [fidx_tbl[pl.ds(g * 16, 16)]], v0)

  cp_l.wait()
  cp_r.wait()

  def loss_j(j, s):
    o = j * _NBINS
    for v in range(_NBINS // 16):
      sl = pl.ds(o + v * 16, 16)
      a_l = acc_l[sl]
      a_r = acc_r[sl]
      lv = lblk[j, pl.ds(v * 16, 16)]
      rv = rblk[j, pl.ds(v * 16, 16)]
      dl = jnp.abs(a_l - lv)
      s = s + jnp.where((dl < 0.2) & (a_l != 0.0), dl, 0.0)
      dr = jnp.abs(a_r - rv)
      s = s + jnp.where((dr < 0.2) & (a_r != 0.0), dr, 0.0)
    return s

  total = lax.fori_loop(0, _JPT, loss_j, jnp.zeros((16,), jnp.float32))
  outv[pl.ds(0, 16)] = total
  pltpu.sync_copy(outv, out_hbm.at[wid])


@functools.partial(
    pl.kernel,
    out_type=jax.ShapeDtypeStruct((_NTILES, 16), jnp.float32),
    mesh=plsc.VectorSubcoreMesh(core_axis_name="c", subcore_axis_name="s"),
    compiler_params=pltpu.CompilerParams(
        use_tc_tiling_on_sc=False, needs_layout_passes=False),
    scratch_types=[
        pltpu.VMEM((_H, _JPT), jnp.float32),
        pltpu.VMEM((_JPT, _NBINS), jnp.float32),
        pltpu.VMEM((_JPT, _NBINS), jnp.float32),
        pltpu.VMEM((_JPT * _NBINS + 16,), jnp.float32),
        pltpu.VMEM((_JPT * _NBINS + 16,), jnp.float32),
        pltpu.VMEM((_H * _JPT,), jnp.int32),
        pltpu.VMEM((16,), jnp.float32),
        pltpu.SemaphoreType.DMA,
        pltpu.SemaphoreType.DMA,
    ],
)
def _sc_loss_partials(up_hbm, left_hbm, right_hbm, out_hbm, *scratch):
  _sc_body(up_hbm, left_hbm, right_hbm, out_hbm, *scratch)


def _tc_reduce_body(p_ref, o_ref):
  o_ref[0, 0] = jnp.sum(p_ref[...]) * (1.0 / float(_B * _H * _W))


@jax.jit
def kernel(up, left, right):
  u2 = up.reshape(_B * _H, _W)
  l2 = left.reshape(_B * _H, _W)
  r2 = right.reshape(_B * _H, _W)
  partials = _sc_loss_partials(u2, l2, r2)
  out = pl.pallas_call(
      _tc_reduce_body,
      out_shape=jax.ShapeDtypeStruct((1, 1), jnp.float32),
      out_specs=pl.BlockSpec(memory_space=pltpu.SMEM),
  )(partials)
  return out[0, 0]


# speedup vs baseline: 1.2038x; 1.0775x over previous
import functools

import jax
import jax.numpy as jnp
from jax import lax
from jax.experimental import pallas as pl
from jax.experimental.pallas import tpu as pltpu
from jax.experimental.pallas import tpu_sc as plsc

_B, _H, _W = 4, 256, 256
_NBINS = 64
_DLO = 104
_NTILES = 32
_JPT = _W // 8
_MAGIC = 12582912.0


def _sc_body(up_hbm, left_hbm, right_hbm, out_hbm,
             ublk, lblk, rblk, acc_l, acc_r, fidx_tbl, outv, sem_u, sem_lr):
  nc = 2
  wid = lax.axis_index("s") * nc + lax.axis_index("c")
  b = wid // 8
  j0 = (wid % 8) * _JPT

  cp_u = pltpu.async_copy(
      up_hbm.at[pl.ds(b * _H, _H), pl.ds(j0, _JPT)], ublk, sem_u)
  cp_l = pltpu.async_copy(
      left_hbm.at[pl.ds(b * _H + j0, _JPT), pl.ds(_DLO, _NBINS)], lblk,
      sem_lr)
  cp_r = pltpu.async_copy(
      right_hbm.at[pl.ds(b * _H + j0, _JPT), pl.ds(_DLO, _NBINS)], rblk,
      sem_lr)

  z = jnp.zeros((16,), jnp.float32)

  def zero_k(k, carry):
    o = k * 16
    acc_l[pl.ds(o, 16)] = z
    acc_r[pl.ds(o, 16)] = z
    return carry

  lax.fori_loop(0, _JPT * _NBINS // 16, zero_k, 0)
  cp_u.wait()

  iota = lax.iota(jnp.int32, 16)
  jbase = (iota * _NBINS, (iota + 16) * _NBINS)
  shift = tuple((_MAGIC + float(_DLO)) - jb.astype(jnp.float32)
                for jb in jbase)
  lo = jbase
  hi = tuple(jb + (_NBINS - 1) for jb in jbase)

  dump = jnp.full((16,), _JPT * _NBINS, jnp.int32) + iota

  @plsc.parallel_loop(0, _H, unroll=4)
  def phase_a(i):
    for g in range(2):
      u = ublk[i, pl.ds(g * 16, 16)]
      su = u * 50.0 + 110.0
      fidx = ((su + _MAGIC) - shift[g]).astype(jnp.int32)
      fidx = jnp.minimum(jnp.maximum(fidx, lo[g]), hi[g])
      fidx_tbl[pl.ds(i * _JPT + g * 16, 16)] = jnp.where(
          u >= 0.0235, fidx, dump)

  def load_pair(k):
    o_r = (k + 129) * _JPT
    o_l = (127 - k) * _JPT
    return (fidx_tbl[pl.ds(o_r, 16)], fidx_tbl[pl.ds(o_r + 16, 16)],
            fidx_tbl[pl.ds(o_l, 16)], fidx_tbl[pl.ds(o_l + 16, 16)])

  def row_pair(k, carry):
    r0, r1, l0, l1 = carry
    nxt = load_pair(k + 1)
    kf = jnp.full((16,), k, jnp.int32).astype(jnp.float32)
    val = (kf + 1.0) / 60.0
    plsc.store_scatter(acc_r, [r0], val)
    plsc.store_scatter(acc_r, [r1], val)
    plsc.store_scatter(acc_l, [l0], val)
    plsc.store_scatter(acc_l, [l1], val)
    return nxt

  lax.fori_loop(0, 127, row_pair, load_pair(0))
  v0 = jnp.full((16,), 128.0, jnp.float32) / 60.0
  for g in range(2):
    plsc.store_scatter(acc_l, [fidx_tbl[pl.ds(g * 16, 16)]], v0)

  cp_l.wait()
  cp_r.wait()

  def loss_j(j, s):
    o = j * _NBINS
    for v in range(_NBINS // 16):
      sl = pl.ds(o + v * 16, 16)
      a_l = acc_l[sl]
      a_r = acc_r[sl]
      lv = lblk[j, pl.ds(v * 16, 16)]
      rv = rblk[j, pl.ds(v * 16, 16)]
      dl = jnp.abs(a_l - lv)
      s = s + jnp.where((dl < 0.2) & (a_l != 0.0), dl, 0.0)
      dr = jnp.abs(a_r - rv)
      s = s + jnp.where((dr < 0.2) & (a_r != 0.0), dr, 0.0)
    return s

  total = lax.fori_loop(0, _JPT, loss_j, jnp.zeros((16,), jnp.float32))
  outv[pl.ds(0, 16)] = total
  pltpu.sync_copy(outv, out_hbm.at[wid])


@functools.partial(
    pl.kernel,
    out_type=jax.ShapeDtypeStruct((_NTILES, 16), jnp.float32),
    mesh=plsc.VectorSubcoreMesh(core_axis_name="c", subcore_axis_name="s"),
    compiler_params=pltpu.CompilerParams(
        use_tc_tiling_on_sc=False, needs_layout_passes=False),
    scratch_types=[
        pltpu.VMEM((_H, _JPT), jnp.float32),
        pltpu.VMEM((_JPT, _NBINS), jnp.float32),
        pltpu.VMEM((_JPT, _NBINS), jnp.float32),
        pltpu.VMEM((_JPT * _NBINS + 16,), jnp.float32),
        pltpu.VMEM((_JPT * _NBINS + 16,), jnp.float32),
        pltpu.VMEM((_H * _JPT + 32,), jnp.int32),
        pltpu.VMEM((16,), jnp.float32),
        pltpu.SemaphoreType.DMA,
        pltpu.SemaphoreType.DMA,
    ],
)
def _sc_loss_partials(up_hbm, left_hbm, right_hbm, out_hbm, *scratch):
  _sc_body(up_hbm, left_hbm, right_hbm, out_hbm, *scratch)


def _tc_reduce_body(p_ref, o_ref):
  o_ref[0, 0] = jnp.sum(p_ref[...]) * (1.0 / float(_B * _H * _W))


@jax.jit
def kernel(up, left, right):
  u2 = up.reshape(_B * _H, _W)
  l2 = left.reshape(_B * _H, _W)
  r2 = right.reshape(_B * _H, _W)
  partials = _sc_loss_partials(u2, l2, r2)
  out = pl.pallas_call(
      _tc_reduce_body,
      out_shape=jax.ShapeDtypeStruct((1, 1), jnp.float32),
      out_specs=pl.BlockSpec(memory_space=pltpu.SMEM),
  )(partials)
  return out[0, 0]
